# Initial kernel scaffold; baseline (speedup 1.0000x reference)
import jax
import jax.numpy as jnp
from jax.experimental import pallas as pl


def kernel(central_node_features, neighbor_node_features, edge_features, Wq, bq, Wk, bk, Wv, bv, We, Wskip, bskip):
    raise NotImplementedError("write your pallas kernel here")


# speedup vs baseline: 8.2051x; 8.2051x over previous
import jax
import jax.numpy as jnp
from jax.experimental import pallas as pl
from jax.experimental.pallas import tpu as pltpu

_BB = 128


def _body(xc_ref, xn_ref, ef_ref, ws_ref, wv_ref, we_ref, bvs_ref, bs_ref,
          out_ref):
    bb, n, d = xn_ref.shape
    c = out_ref.shape[-1]
    xc = xc_ref[...]
    ws = ws_ref[...]
    center = jnp.dot(xc, ws, preferred_element_type=jnp.float32) + bs_ref[...]
    vc = jnp.dot(xc, wv_ref[...], preferred_element_type=jnp.float32) + bvs_ref[...]
    xn = xn_ref[...].reshape(bb * n, d)
    ef = ef_ref[...].reshape(bb * n, d)
    nbr = jnp.dot(xn, ws, preferred_element_type=jnp.float32)
    nbr = nbr + jnp.dot(ef, we_ref[...], preferred_element_type=jnp.float32)
    outn = nbr.reshape(bb, n, c) + vc[:, None, :]
    out_ref[...] = jnp.concatenate([center[:, None, :], outn], axis=1)


def kernel(central_node_features, neighbor_node_features, edge_features,
           Wq, bq, Wk, bk, Wv, bv, We, Wskip, bskip):
    b, n, d = neighbor_node_features.shape
    c = Wskip.shape[0]
    xc = central_node_features.reshape(b, d)
    ws_t = Wskip.T
    wv_t = Wv.T
    we_t = We.T
    bvs = (bv + bskip).reshape(1, c)
    bs = bskip.reshape(1, c)

    out = pl.pallas_call(
        _body,
        grid=(b // _BB,),
        in_specs=[
            pl.BlockSpec((_BB, d), lambda i: (i, 0)),
            pl.BlockSpec((_BB, n, d), lambda i: (i, 0, 0)),
            pl.BlockSpec((_BB, n, d), lambda i: (i, 0, 0)),
            pl.BlockSpec((d, c), lambda i: (0, 0)),
            pl.BlockSpec((d, c), lambda i: (0, 0)),
            pl.BlockSpec((d, c), lambda i: (0, 0)),
            pl.BlockSpec((1, c), lambda i: (0, 0)),
            pl.BlockSpec((1, c), lambda i: (0, 0)),
        ],
        out_specs=pl.BlockSpec((_BB, n + 1, c), lambda i: (i, 0, 0)),
        out_shape=jax.ShapeDtypeStruct((b, n + 1, c), jnp.float32),
        compiler_params=pltpu.CompilerParams(
            dimension_semantics=("arbitrary",)),
    )(xc, neighbor_node_features, edge_features, ws_t, wv_t, we_t, bvs, bs)
    return out.reshape(b * (n + 1), c)
